# baseline (speedup 1.0000x reference)
import functools

import numpy as np
import jax
import jax.numpy as jnp
from jax import lax
from jax.experimental import pallas as pl
from jax.experimental.pallas import tpu as pltpu
from jax.experimental.pallas import tpu_sc as plsc

_B = 4
_N = 5000
_L = 16
_NP = 5024
_NCH = _NP // _L
_NCH2 = _NP // (2 * _L)
_MAXOUT = 300
_IOU_THR = 0.6
_SCORE_THR = 0.01
_NEG_INF = float("-inf")
_TARGET = 448
_BS_ITERS = 16
_DEADIDX = _NP - 1
_THR_BITS = int(np.float32(_SCORE_THR).view(np.int32))
_INF_BITS = int(np.float32(np.inf).view(np.int32))
_FMIN = float(np.finfo(np.float32).min)

_mesh = plsc.VectorSubcoreMesh(core_axis_name="c", subcore_axis_name="s")


@functools.partial(
    pl.kernel,
    out_type=jax.ShapeDtypeStruct((_B, _MAXOUT * 8), jnp.float32),
    mesh=_mesh,
    compiler_params=pltpu.CompilerParams(needs_layout_passes=False),
    scratch_types=[
        pltpu.VMEM((_NP,), jnp.float32),
        pltpu.VMEM((_NP,), jnp.float32),
        pltpu.VMEM((_NP,), jnp.float32),
        pltpu.VMEM((_NP,), jnp.float32),
        pltpu.VMEM((_NP,), jnp.float32),
        pltpu.VMEM((_NP,), jnp.float32),
        pltpu.VMEM((_NP,), jnp.float32),
        pltpu.VMEM((_NP,), jnp.float32),
        pltpu.VMEM((_NP,), jnp.float32),
        pltpu.VMEM((_NP,), jnp.float32),
        pltpu.VMEM((_NP,), jnp.float32),
        pltpu.VMEM((_NP + _L,), jnp.int32),
        pltpu.VMEM((_MAXOUT + 4,), jnp.int32),
        pltpu.VMEM((_L,), jnp.float32),
        pltpu.VMEM((_L,), jnp.int32),
        pltpu.VMEM((_MAXOUT * 8,), jnp.float32),
    ],
)
def _nms_sc(x1h, y1h, x2h, y2h, sch, labh, outh,
            x1, y1, x2, y2, sc, labf, bx1, by1, bx2, by2, area,
            idxlist, wlist, redv, redi, outbuf):
    cid = lax.axis_index("c")
    sid = lax.axis_index("s")
    wid = sid * 2 + cid

    @pl.when(wid < _B)
    def _():
        img = wid
        pltpu.sync_copy(x1h.at[img], x1)
        pltpu.sync_copy(y1h.at[img], y1)
        pltpu.sync_copy(x2h.at[img], x2)
        pltpu.sync_copy(y2h.at[img], y2)
        pltpu.sync_copy(sch.at[img], sc)
        pltpu.sync_copy(labh.at[img], labf)

        ninf = jnp.full((_L,), _NEG_INF, jnp.float32)
        zidx = jnp.zeros((_L,), jnp.int32)
        lane = lax.iota(jnp.int32, _L)
        i32 = jnp.int32

        def lane_max(v):
            for st in (1, 2, 4, 8):
                redv[...] = v
                v = jnp.maximum(v, plsc.load_gather(redv, [lane ^ st]))
            return v

        def lane_sum_i32(v):
            for st in (1, 2, 4, 8):
                redi[...] = v
                v = v + plsc.load_gather(redi, [lane ^ st])
            return v

        def lane_argmax(v, i):
            for st in (1, 2, 4, 8):
                redv[...] = v
                redi[...] = i
                perm = lane ^ st
                gv = plsc.load_gather(redv, [perm])
                gi = plsc.load_gather(redi, [perm])
                take = (gv > v) | ((gv == v) & (gi < i))
                v = jnp.where(take, gv, v)
                i = jnp.where(take, gi, i)
            return v, i

        def splat_to_scalar(v, nbits):
            tot = i32(0)
            for b in range(nbits):
                bit = jnp.any((v & i32(1 << b)) != i32(0))
                tot = tot + jnp.where(bit, i32(1 << b), i32(0))
            return tot

        def mc_body(k, acc):
            for u in range(2):
                b = (2 * k + u) * _L
                acc = jnp.maximum(
                    jnp.maximum(acc, jnp.maximum(x1[pl.ds(b, _L)],
                                                 y1[pl.ds(b, _L)])),
                    jnp.maximum(x2[pl.ds(b, _L)], y2[pl.ds(b, _L)]))
            return acc

        mcp1 = lane_max(lax.fori_loop(0, _NCH2, mc_body, ninf)) + jnp.float32(1.0)

        def stage_body(k, carry):
            for u in range(2):
                b = (2 * k + u) * _L
                l = labf[pl.ds(b, _L)] * mcp1
                a1 = x1[pl.ds(b, _L)] + l
                a2 = y1[pl.ds(b, _L)] + l
                a3 = x2[pl.ds(b, _L)] + l
                a4 = y2[pl.ds(b, _L)] + l
                bx1[pl.ds(b, _L)] = a1
                by1[pl.ds(b, _L)] = a2
                bx2[pl.ds(b, _L)] = a3
                by2[pl.ds(b, _L)] = a4
                area[pl.ds(b, _L)] = (a3 - a1) * (a4 - a2)
                s0 = sc[pl.ds(b, _L)]
                sc[pl.ds(b, _L)] = jnp.where(s0 >= _SCORE_THR, s0, _NEG_INF)
            return carry

        lax.fori_loop(0, _NCH2, stage_body, i32(0))

        def bs_body(_, c):
            lo, hi, best = c
            mid = (lo + hi) >> 1
            tv = lax.bitcast_convert_type(mid, jnp.float32)

            def cnt_body(kk, acc):
                for u in range(2):
                    b = (2 * kk + u) * _L
                    acc = acc + jnp.where(sc[pl.ds(b, _L)] >= tv,
                                          i32(1), i32(0))
                return acc

            cntl = lax.fori_loop(0, _NCH2, cnt_body, zidx)
            ok = lane_sum_i32(cntl) >= i32(_TARGET)
            best = jnp.where(ok, mid, best)
            lo = jnp.where(ok, mid + 1, lo)
            hi = jnp.where(ok, hi, mid)
            return lo, hi, best

        lob = jnp.full((_L,), i32(_THR_BITS), i32)
        hib = jnp.full((_L,), i32(_INF_BITS), i32)
        _, _, bestb = lax.fori_loop(0, _BS_ITERS, bs_body, (lob, hib, lob))
        thr0 = lax.bitcast_convert_type(bestb, jnp.float32)

        def compact(thr_v):
            def body(k, c):
                off, cbv, cbi = c
                b = k * _L
                s0 = sc[pl.ds(b, _L)]
                mask = s0 >= thr_v
                cnt = plsc.all_reduce_population_count(mask)
                pos = plsc.cumsum(jnp.where(mask, i32(1), i32(0)))
                idxv = b + lane
                plsc.store_scatter(idxlist, [off + pos - 1], idxv, mask=mask)
                sm = jnp.where(mask, s0, ninf)
                upd = sm > cbv
                return (off + cnt, jnp.where(upd, sm, cbv),
                        jnp.where(upd, idxv, cbi))

            off, cbv, cbi = lax.fori_loop(0, _NCH, body, (zidx, ninf, zidx))
            plsc.store_scatter(idxlist, [off + lane],
                               jnp.full((_L,), i32(_DEADIDX), i32))
            ncc = splat_to_scalar((off + i32(_L - 1)) >> 4, 10)
            return ncc, cbv, cbi

        ncc0, bv0, bi0 = compact(thr0)

        def iter_body(t, carry):
            bv, bi, ncc, nw, exhausted = carry

            need_rebuild = jnp.logical_not(jnp.any(bv > _NEG_INF)) \
                & jnp.logical_not(exhausted)

            def rebuild(_):
                nccr, _, _ = compact(jnp.full((_L,), _FMIN, jnp.float32))

                def replay_w(j, carry2):
                    wi = plsc.load_gather(wlist, [jnp.full((_L,), j, i32)])
                    wx1 = plsc.load_gather(bx1, [wi])
                    wy1 = plsc.load_gather(by1, [wi])
                    wx2 = plsc.load_gather(bx2, [wi])
                    wy2 = plsc.load_gather(by2, [wi])
                    war = plsc.load_gather(area, [wi])

                    def rsweep(k, c3):
                        b = k * _L
                        ji = idxlist[pl.ds(b, _L)]
                        s0 = plsc.load_gather(sc, [ji])
                        q1 = plsc.load_gather(bx1, [ji])
                        q2 = plsc.load_gather(by1, [ji])
                        q3 = plsc.load_gather(bx2, [ji])
                        q4 = plsc.load_gather(by2, [ji])
                        qa = plsc.load_gather(area, [ji])
                        xx1 = jnp.maximum(wx1, q1)
                        yy1 = jnp.maximum(wy1, q2)
                        xx2 = jnp.minimum(wx2, q3)
                        yy2 = jnp.minimum(wy2, q4)
                        w = jnp.maximum(xx2 - xx1, jnp.float32(0.0))
                        h = jnp.maximum(yy2 - yy1, jnp.float32(0.0))
                        inter = w * h
                        iou = inter / (war + qa - inter + jnp.float32(1e-9))
                        plsc.store_scatter(
                            sc, [ji],
                            jnp.where(iou > _IOU_THR, _NEG_INF, s0))
                        return c3

                    lax.fori_loop(0, nccr, rsweep, i32(0))
                    return carry2

                lax.fori_loop(0, nw, replay_w, i32(0))

                def amax_body(k, c3):
                    abv, abi = c3
                    b = k * _L
                    ji = idxlist[pl.ds(b, _L)]
                    s0 = plsc.load_gather(sc, [ji])
                    upd = s0 > abv
                    return jnp.where(upd, s0, abv), jnp.where(upd, ji, abi)

                rbv, rbi = lax.fori_loop(0, nccr, amax_body, (ninf, zidx))
                exh = jnp.logical_not(jnp.any(rbv > _NEG_INF))
                return rbv, rbi, nccr, exh

            def norebuild(_):
                return bv, bi, ncc, exhausted

            bv, bi, ncc, exhausted = lax.cond(
                need_rebuild, rebuild, norebuild, 0)

            mv, iv = lane_argmax(bv, bi)
            validv = mv > _NEG_INF

            g1 = plsc.load_gather(x1, [iv])
            g2 = plsc.load_gather(y1, [iv])
            g3 = plsc.load_gather(x2, [iv])
            g4 = plsc.load_gather(y2, [iv])
            gl = plsc.load_gather(labf, [iv])
            row = jnp.where(lane == 0, g1,
                  jnp.where(lane == 1, g2,
                  jnp.where(lane == 2, g3,
                  jnp.where(lane == 3, g4,
                  jnp.where(lane == 4, mv,
                  jnp.where(lane == 5, gl, jnp.float32(0.0)))))))
            row = jnp.where(validv, row, jnp.float32(0.0))
            plsc.store_scatter(outbuf, [t * 8 + lane], row, mask=lane < 8)

            plsc.store_scatter(wlist, [jnp.full((_L,), nw, i32)], iv,
                               mask=validv & (lane == 0))
            nw = nw + jnp.where(jnp.any(validv), i32(1), i32(0))

            sx1 = plsc.load_gather(bx1, [iv])
            sy1 = plsc.load_gather(by1, [iv])
            sx2 = plsc.load_gather(bx2, [iv])
            sy2 = plsc.load_gather(by2, [iv])
            sar = plsc.load_gather(area, [iv])

            def sup_body(k, c2):
                nbv, nbi = c2
                b = k * _L
                ji = idxlist[pl.ds(b, _L)]
                s0 = plsc.load_gather(sc, [ji])
                q1 = plsc.load_gather(bx1, [ji])
                q2 = plsc.load_gather(by1, [ji])
                q3 = plsc.load_gather(bx2, [ji])
                q4 = plsc.load_gather(by2, [ji])
                qa = plsc.load_gather(area, [ji])
                xx1 = jnp.maximum(sx1, q1)
                yy1 = jnp.maximum(sy1, q2)
                xx2 = jnp.minimum(sx2, q3)
                yy2 = jnp.minimum(sy2, q4)
                w = jnp.maximum(xx2 - xx1, jnp.float32(0.0))
                h = jnp.maximum(yy2 - yy1, jnp.float32(0.0))
                inter = w * h
                iou = inter / (sar + qa - inter + jnp.float32(1e-9))
                s0n = jnp.where(iou > _IOU_THR, _NEG_INF, s0)
                plsc.store_scatter(sc, [ji], s0n)
                upd = s0n > nbv
                return jnp.where(upd, s0n, nbv), jnp.where(upd, ji, nbi)

            nbv, nbi = lax.fori_loop(0, ncc, sup_body, (ninf, zidx))
            return nbv, nbi, ncc, nw, exhausted

        lax.fori_loop(0, _MAXOUT, iter_body,
                      (bv0, bi0, ncc0, i32(0), jnp.bool_(False)))
        pltpu.sync_copy(outbuf, outh.at[img])


def kernel(boxes, scores, labels):
    pad = _NP - _N

    def padr(a, v):
        return jnp.pad(a, ((0, 0), (0, pad)), constant_values=v)

    x1p = padr(boxes[..., 0], 0.0)
    y1p = padr(boxes[..., 1], 0.0)
    x2p = padr(boxes[..., 2], 0.0)
    y2p = padr(boxes[..., 3], 0.0)
    scp = padr(scores, 0.0)
    labp = padr(labels.astype(jnp.float32), 0.0)
    out = _nms_sc(x1p, y1p, x2p, y2p, scp, labp)
    return out.reshape(_B, _MAXOUT, 8)[:, :, :6]


# speedup vs baseline: 5.1554x; 5.1554x over previous
import functools

import numpy as np
import jax
import jax.numpy as jnp
from jax import lax
from jax.experimental import pallas as pl
from jax.experimental.pallas import tpu as pltpu
from jax.experimental.pallas import tpu_sc as plsc

_B = 4
_N = 5000
_L = 16
_NP = 5024
_NCH = _NP // _L
_NCH2 = _NP // (2 * _L)
_MAXOUT = 300
_IOU_THR = 0.6
_SCORE_THR = 0.01
_NEG_INF = float("-inf")
_TARGET = 448
_CMAX = 576
_CCH2 = _CMAX // (2 * _L)
_BS_ITERS = 18
_DEADIDX = _NP - 1
_THR_BITS = int(np.float32(_SCORE_THR).view(np.int32))
_INF_BITS = int(np.float32(np.inf).view(np.int32))
_FMIN = float(np.finfo(np.float32).min)

_mesh = plsc.VectorSubcoreMesh(core_axis_name="c", subcore_axis_name="s")


@functools.partial(
    pl.kernel,
    out_type=jax.ShapeDtypeStruct((_B, _MAXOUT * 8), jnp.float32),
    mesh=_mesh,
    compiler_params=pltpu.CompilerParams(needs_layout_passes=False),
    scratch_types=[
        pltpu.VMEM((_NP,), jnp.float32),
        pltpu.VMEM((_NP,), jnp.float32),
        pltpu.VMEM((_NP,), jnp.float32),
        pltpu.VMEM((_NP,), jnp.float32),
        pltpu.VMEM((_NP,), jnp.float32),
        pltpu.VMEM((_NP,), jnp.float32),
        pltpu.VMEM((_NP,), jnp.float32),
        pltpu.VMEM((_NP,), jnp.float32),
        pltpu.VMEM((_NP,), jnp.float32),
        pltpu.VMEM((_NP,), jnp.float32),
        pltpu.VMEM((_NP,), jnp.float32),
        pltpu.VMEM((_CMAX,), jnp.float32),
        pltpu.VMEM((_CMAX,), jnp.float32),
        pltpu.VMEM((_CMAX,), jnp.float32),
        pltpu.VMEM((_CMAX,), jnp.float32),
        pltpu.VMEM((_CMAX,), jnp.float32),
        pltpu.VMEM((_CMAX,), jnp.float32),
        pltpu.VMEM((_CMAX,), jnp.int32),
        pltpu.VMEM((_MAXOUT + 4,), jnp.int32),
        pltpu.VMEM((_L,), jnp.float32),
        pltpu.VMEM((_L,), jnp.int32),
        pltpu.VMEM((_MAXOUT * 8,), jnp.float32),
    ],
)
def _nms_sc(x1h, y1h, x2h, y2h, sch, labh, outh,
            x1, y1, x2, y2, sc, labf, bx1, by1, bx2, by2, area,
            cx1, cy1, cx2, cy2, car, csc, corig,
            wlist, redv, redi, outbuf):
    cid = lax.axis_index("c")
    sid = lax.axis_index("s")
    wid = sid * 2 + cid

    @pl.when(wid < _B)
    def _():
        img = wid
        pltpu.sync_copy(x1h.at[img], x1)
        pltpu.sync_copy(y1h.at[img], y1)
        pltpu.sync_copy(x2h.at[img], x2)
        pltpu.sync_copy(y2h.at[img], y2)
        pltpu.sync_copy(sch.at[img], sc)
        pltpu.sync_copy(labh.at[img], labf)

        ninf = jnp.full((_L,), _NEG_INF, jnp.float32)
        zidx = jnp.zeros((_L,), jnp.int32)
        zf = jnp.zeros((_L,), jnp.float32)
        lane = lax.iota(jnp.int32, _L)
        i32 = jnp.int32

        def lane_sum_i32(v):
            for st in (1, 2, 4, 8):
                redi[...] = v
                v = v + plsc.load_gather(redi, [lane ^ st])
            return v

        def lane_max(v):
            for st in (1, 2, 4, 8):
                redv[...] = v
                v = jnp.maximum(v, plsc.load_gather(redv, [lane ^ st]))
            return v

        def lane_argmax(v, i):
            for st in (1, 2, 4, 8):
                redv[...] = v
                redi[...] = i
                perm = lane ^ st
                gv = plsc.load_gather(redv, [perm])
                gi = plsc.load_gather(redi, [perm])
                take = (gv > v) | ((gv == v) & (gi < i))
                v = jnp.where(take, gv, v)
                i = jnp.where(take, gi, i)
            return v, i

        def mc_body(k, acc):
            for u in range(2):
                b = (2 * k + u) * _L
                acc = jnp.maximum(
                    jnp.maximum(acc, jnp.maximum(x1[pl.ds(b, _L)],
                                                 y1[pl.ds(b, _L)])),
                    jnp.maximum(x2[pl.ds(b, _L)], y2[pl.ds(b, _L)]))
            return acc

        mcp1 = lane_max(lax.fori_loop(0, _NCH2, mc_body, ninf)) + jnp.float32(1.0)

        def stage_body(k, carry):
            for u in range(2):
                b = (2 * k + u) * _L
                l = labf[pl.ds(b, _L)] * mcp1
                a1 = x1[pl.ds(b, _L)] + l
                a2 = y1[pl.ds(b, _L)] + l
                a3 = x2[pl.ds(b, _L)] + l
                a4 = y2[pl.ds(b, _L)] + l
                bx1[pl.ds(b, _L)] = a1
                by1[pl.ds(b, _L)] = a2
                bx2[pl.ds(b, _L)] = a3
                by2[pl.ds(b, _L)] = a4
                area[pl.ds(b, _L)] = (a3 - a1) * (a4 - a2)
                s0 = sc[pl.ds(b, _L)]
                sc[pl.ds(b, _L)] = jnp.where(s0 >= _SCORE_THR, s0, _NEG_INF)
            return carry

        lax.fori_loop(0, _NCH2, stage_body, i32(0))

        def bs_body(_, c):
            lo, hi, best = c
            mid = (lo + hi) >> 1
            tv = lax.bitcast_convert_type(mid, jnp.float32)

            def cnt_body(kk, acc):
                for u in range(2):
                    b = (2 * kk + u) * _L
                    acc = acc + jnp.where(sc[pl.ds(b, _L)] >= tv,
                                          i32(1), i32(0))
                return acc

            cntl = lax.fori_loop(0, _NCH2, cnt_body, zidx)
            ok = lane_sum_i32(cntl) >= i32(_TARGET)
            best = jnp.where(ok, mid, best)
            lo = jnp.where(ok, mid + 1, lo)
            hi = jnp.where(ok, hi, mid)
            return lo, hi, best

        lob = jnp.full((_L,), i32(_THR_BITS), i32)
        hib = jnp.full((_L,), i32(_INF_BITS), i32)
        _, _, bestb = lax.fori_loop(0, _BS_ITERS, bs_body, (lob, hib, lob))
        thr0 = lax.bitcast_convert_type(bestb, jnp.float32)

        def fill_body(k, carry):
            for u in range(2):
                b = (2 * k + u) * _L
                cx1[pl.ds(b, _L)] = zf
                cy1[pl.ds(b, _L)] = zf
                cx2[pl.ds(b, _L)] = zf
                cy2[pl.ds(b, _L)] = zf
                car[pl.ds(b, _L)] = zf
                csc[pl.ds(b, _L)] = ninf
                corig[pl.ds(b, _L)] = jnp.full((_L,), i32(_DEADIDX), i32)
            return carry

        lax.fori_loop(0, _CCH2, fill_body, i32(0))

        def comp_body(k, c):
            off, cbv, cbi = c
            b = k * _L
            s0 = sc[pl.ds(b, _L)]
            mask = s0 >= thr0
            cnt = plsc.all_reduce_population_count(mask)
            pos = off + plsc.cumsum(jnp.where(mask, i32(1), i32(0))) - 1
            maskw = mask & (pos < i32(_CMAX))
            idxv = b + lane
            plsc.store_scatter(csc, [pos], s0, mask=maskw)
            plsc.store_scatter(cx1, [pos], bx1[pl.ds(b, _L)], mask=maskw)
            plsc.store_scatter(cy1, [pos], by1[pl.ds(b, _L)], mask=maskw)
            plsc.store_scatter(cx2, [pos], bx2[pl.ds(b, _L)], mask=maskw)
            plsc.store_scatter(cy2, [pos], by2[pl.ds(b, _L)], mask=maskw)
            plsc.store_scatter(car, [pos], area[pl.ds(b, _L)], mask=maskw)
            plsc.store_scatter(corig, [pos], idxv, mask=maskw)
            sm = jnp.where(maskw, s0, ninf)
            upd = sm > cbv
            return (off + cnt, jnp.where(upd, sm, cbv),
                    jnp.where(upd, pos, cbi))

        offv, cbv0, cbi0 = lax.fori_loop(0, _NCH, comp_body,
                                         (zidx, ninf, zidx))
        overflow = jnp.any(offv > i32(_CMAX))

        def famax_body(k, c):
            fbv, fbi = c
            for u in range(2):
                b = (2 * k + u) * _L
                s0 = sc[pl.ds(b, _L)]
                upd = s0 > fbv
                fbv = jnp.where(upd, s0, fbv)
                fbi = jnp.where(upd, b + lane, fbi)
            return fbv, fbi

        def init_full(_):
            fbv, fbi = lax.fori_loop(0, _NCH2, famax_body, (ninf, zidx))
            return fbv, fbi, i32(1)

        def init_compact(_):
            return cbv0, cbi0, i32(0)

        bv0, bi0, mode0 = lax.cond(overflow, init_full, init_compact, 0)

        def kill_full(wx1, wy1, wx2, wy2, war):
            def kbody(k, carry):
                for u in range(2):
                    b = (2 * k + u) * _L
                    s0 = sc[pl.ds(b, _L)]
                    xx1 = jnp.maximum(wx1, bx1[pl.ds(b, _L)])
                    yy1 = jnp.maximum(wy1, by1[pl.ds(b, _L)])
                    xx2 = jnp.minimum(wx2, bx2[pl.ds(b, _L)])
                    yy2 = jnp.minimum(wy2, by2[pl.ds(b, _L)])
                    w = jnp.maximum(xx2 - xx1, jnp.float32(0.0))
                    h = jnp.maximum(yy2 - yy1, jnp.float32(0.0))
                    inter = w * h
                    iou = inter / (war + area[pl.ds(b, _L)] - inter
                                   + jnp.float32(1e-9))
                    sc[pl.ds(b, _L)] = jnp.where(iou > _IOU_THR,
                                                 _NEG_INF, s0)
                return carry

            lax.fori_loop(0, _NCH2, kbody, i32(0))

        def iter_body(t, carry):
            bv, bi, nw, mode = carry

            trans = (mode == i32(0)) & jnp.logical_not(
                jnp.any(bv > _NEG_INF))

            def do_trans(_):
                def replay_w(j, carry2):
                    wi = plsc.load_gather(wlist, [jnp.full((_L,), j, i32)])
                    kill_full(plsc.load_gather(bx1, [wi]),
                              plsc.load_gather(by1, [wi]),
                              plsc.load_gather(bx2, [wi]),
                              plsc.load_gather(by2, [wi]),
                              plsc.load_gather(area, [wi]))
                    return carry2

                lax.fori_loop(0, nw, replay_w, i32(0))
                fbv, fbi = lax.fori_loop(0, _NCH2, famax_body, (ninf, zidx))
                return fbv, fbi, i32(1)

            bv, bi, mode = lax.cond(trans, do_trans,
                                    lambda _: (bv, bi, mode), 0)
            cmode = mode == i32(0)

            mv, ivl = lane_argmax(bv, bi)
            validv = mv > _NEG_INF

            io_c = plsc.load_gather(corig, [jnp.where(cmode, ivl, zidx)])
            iorig = jnp.where(cmode, io_c, ivl)

            g1 = plsc.load_gather(x1, [iorig])
            g2 = plsc.load_gather(y1, [iorig])
            g3 = plsc.load_gather(x2, [iorig])
            g4 = plsc.load_gather(y2, [iorig])
            gl = plsc.load_gather(labf, [iorig])
            row = jnp.where(lane == 0, g1,
                  jnp.where(lane == 1, g2,
                  jnp.where(lane == 2, g3,
                  jnp.where(lane == 3, g4,
                  jnp.where(lane == 4, mv,
                  jnp.where(lane == 5, gl, jnp.float32(0.0)))))))
            row = jnp.where(validv, row, jnp.float32(0.0))
            plsc.store_scatter(outbuf, [t * 8 + lane], row, mask=lane < 8)

            plsc.store_scatter(wlist, [jnp.full((_L,), nw, i32)], iorig,
                               mask=validv & (lane == 0))
            nw = nw + jnp.where(jnp.any(validv), i32(1), i32(0))

            sx1 = plsc.load_gather(bx1, [iorig])
            sy1 = plsc.load_gather(by1, [iorig])
            sx2 = plsc.load_gather(bx2, [iorig])
            sy2 = plsc.load_gather(by2, [iorig])
            sar = plsc.load_gather(area, [iorig])

            def csweep(_):
                def body(k, c2):
                    nbv, nbi = c2
                    for u in range(2):
                        b = (2 * k + u) * _L
                        s0 = csc[pl.ds(b, _L)]
                        xx1 = jnp.maximum(sx1, cx1[pl.ds(b, _L)])
                        yy1 = jnp.maximum(sy1, cy1[pl.ds(b, _L)])
                        xx2 = jnp.minimum(sx2, cx2[pl.ds(b, _L)])
                        yy2 = jnp.minimum(sy2, cy2[pl.ds(b, _L)])
                        w = jnp.maximum(xx2 - xx1, jnp.float32(0.0))
                        h = jnp.maximum(yy2 - yy1, jnp.float32(0.0))
                        inter = w * h
                        iou = inter / (sar + car[pl.ds(b, _L)] - inter
                                       + jnp.float32(1e-9))
                        s0n = jnp.where(iou > _IOU_THR, _NEG_INF, s0)
                        csc[pl.ds(b, _L)] = s0n
                        upd = s0n > nbv
                        nbv = jnp.where(upd, s0n, nbv)
                        nbi = jnp.where(upd, b + lane, nbi)
                    return nbv, nbi

                return lax.fori_loop(0, _CCH2, body, (ninf, zidx))

            def fsweep(_):
                def body(k, c2):
                    nbv, nbi = c2
                    for u in range(2):
                        b = (2 * k + u) * _L
                        s0 = sc[pl.ds(b, _L)]
                        xx1 = jnp.maximum(sx1, bx1[pl.ds(b, _L)])
                        yy1 = jnp.maximum(sy1, by1[pl.ds(b, _L)])
                        xx2 = jnp.minimum(sx2, bx2[pl.ds(b, _L)])
                        yy2 = jnp.minimum(sy2, by2[pl.ds(b, _L)])
                        w = jnp.maximum(xx2 - xx1, jnp.float32(0.0))
                        h = jnp.maximum(yy2 - yy1, jnp.float32(0.0))
                        inter = w * h
                        iou = inter / (sar + area[pl.ds(b, _L)] - inter
                                       + jnp.float32(1e-9))
                        s0n = jnp.where(iou > _IOU_THR, _NEG_INF, s0)
                        sc[pl.ds(b, _L)] = s0n
                        upd = s0n > nbv
                        nbv = jnp.where(upd, s0n, nbv)
                        nbi = jnp.where(upd, b + lane, nbi)
                    return nbv, nbi

                return lax.fori_loop(0, _NCH2, body, (ninf, zidx))

            nbv, nbi = lax.cond(cmode, csweep, fsweep, 0)
            return nbv, nbi, nw, mode

        lax.fori_loop(0, _MAXOUT, iter_body, (bv0, bi0, i32(0), mode0))
        pltpu.sync_copy(outbuf, outh.at[img])


def kernel(boxes, scores, labels):
    pad = _NP - _N

    def padr(a, v):
        return jnp.pad(a, ((0, 0), (0, pad)), constant_values=v)

    x1p = padr(boxes[..., 0], 0.0)
    y1p = padr(boxes[..., 1], 0.0)
    x2p = padr(boxes[..., 2], 0.0)
    y2p = padr(boxes[..., 3], 0.0)
    scp = padr(scores, 0.0)
    labp = padr(labels.astype(jnp.float32), 0.0)
    out = _nms_sc(x1p, y1p, x2p, y2p, scp, labp)
    return out.reshape(_B, _MAXOUT, 8)[:, :, :6]


# speedup vs baseline: 5.2923x; 1.0266x over previous
import functools

import numpy as np
import jax
import jax.numpy as jnp
from jax import lax
from jax.experimental import pallas as pl
from jax.experimental.pallas import tpu as pltpu
from jax.experimental.pallas import tpu_sc as plsc

_B = 4
_N = 5000
_L = 16
_NP = 5024
_NCH = _NP // _L
_NCH2 = _NP // (2 * _L)
_MAXOUT = 300
_IOU_THR = 0.6
_SCORE_THR = 0.01
_NEG_INF = float("-inf")
_TARGET = 448
_CMAX = 576
_CCH2 = _CMAX // (2 * _L)
_BS_ITERS = 18
_DEADIDX = _NP - 1
_BIG = 1 << 30
_THR_BITS = int(np.float32(_SCORE_THR).view(np.int32))
_INF_BITS = int(np.float32(np.inf).view(np.int32))

_mesh = plsc.VectorSubcoreMesh(core_axis_name="c", subcore_axis_name="s")


@functools.partial(
    pl.kernel,
    out_type=jax.ShapeDtypeStruct((_B, _MAXOUT * 8), jnp.float32),
    mesh=_mesh,
    compiler_params=pltpu.CompilerParams(needs_layout_passes=False),
    scratch_types=[
        pltpu.VMEM((_NP,), jnp.float32),
        pltpu.VMEM((_NP,), jnp.float32),
        pltpu.VMEM((_NP,), jnp.float32),
        pltpu.VMEM((_NP,), jnp.float32),
        pltpu.VMEM((_NP,), jnp.float32),
        pltpu.VMEM((_NP,), jnp.float32),
        pltpu.VMEM((_NP,), jnp.float32),
        pltpu.VMEM((_NP,), jnp.float32),
        pltpu.VMEM((_NP,), jnp.float32),
        pltpu.VMEM((_NP,), jnp.float32),
        pltpu.VMEM((_NP,), jnp.float32),
        pltpu.VMEM((_CMAX,), jnp.float32),
        pltpu.VMEM((_CMAX,), jnp.float32),
        pltpu.VMEM((_CMAX,), jnp.float32),
        pltpu.VMEM((_CMAX,), jnp.float32),
        pltpu.VMEM((_CMAX,), jnp.float32),
        pltpu.VMEM((_CMAX,), jnp.float32),
        pltpu.VMEM((_CMAX,), jnp.int32),
        pltpu.VMEM((_MAXOUT + 4,), jnp.int32),
        pltpu.VMEM((_MAXOUT * 8,), jnp.float32),
    ],
)
def _nms_sc(x1h, y1h, x2h, y2h, sch, labh, outh,
            x1, y1, x2, y2, sc, labf, bx1, by1, bx2, by2, area,
            cx1, cy1, cx2, cy2, car, csc, corig,
            wlist, outbuf):
    cid = lax.axis_index("c")
    sid = lax.axis_index("s")
    wid = sid * 2 + cid

    @pl.when(wid < _B)
    def _():
        img = wid
        pltpu.sync_copy(x1h.at[img], x1)
        pltpu.sync_copy(y1h.at[img], y1)
        pltpu.sync_copy(x2h.at[img], x2)
        pltpu.sync_copy(y2h.at[img], y2)
        pltpu.sync_copy(sch.at[img], sc)
        pltpu.sync_copy(labh.at[img], labf)

        ninf = jnp.full((_L,), _NEG_INF, jnp.float32)
        zidx = jnp.zeros((_L,), jnp.int32)
        zf = jnp.zeros((_L,), jnp.float32)
        lane = lax.iota(jnp.int32, _L)
        i32 = jnp.int32

        def mc_body(k, acc):
            for u in range(2):
                b = (2 * k + u) * _L
                acc = jnp.maximum(
                    jnp.maximum(acc, jnp.maximum(x1[pl.ds(b, _L)],
                                                 y1[pl.ds(b, _L)])),
                    jnp.maximum(x2[pl.ds(b, _L)], y2[pl.ds(b, _L)]))
            return acc

        mcp1 = jnp.max(lax.fori_loop(0, _NCH2, mc_body, ninf)) \
            + jnp.float32(1.0)

        def stage_body(k, carry):
            for u in range(2):
                b = (2 * k + u) * _L
                l = labf[pl.ds(b, _L)] * mcp1
                a1 = x1[pl.ds(b, _L)] + l
                a2 = y1[pl.ds(b, _L)] + l
                a3 = x2[pl.ds(b, _L)] + l
                a4 = y2[pl.ds(b, _L)] + l
                bx1[pl.ds(b, _L)] = a1
                by1[pl.ds(b, _L)] = a2
                bx2[pl.ds(b, _L)] = a3
                by2[pl.ds(b, _L)] = a4
                area[pl.ds(b, _L)] = (a3 - a1) * (a4 - a2)
                s0 = sc[pl.ds(b, _L)]
                sc[pl.ds(b, _L)] = jnp.where(s0 >= _SCORE_THR, s0, _NEG_INF)
            return carry

        lax.fori_loop(0, _NCH2, stage_body, i32(0))

        def bs_body(_, c):
            lo, hi, best = c
            mid = (lo + hi) >> 1
            tv = lax.bitcast_convert_type(jnp.full((_L,), mid, i32),
                                          jnp.float32)

            def cnt_body(kk, acc):
                for u in range(2):
                    b = (2 * kk + u) * _L
                    acc = acc + jnp.where(sc[pl.ds(b, _L)] >= tv,
                                          i32(1), i32(0))
                return acc

            cnt = jnp.sum(lax.fori_loop(0, _NCH2, cnt_body, zidx))
            ok = cnt >= i32(_TARGET)
            best = jnp.where(ok, mid, best)
            lo = jnp.where(ok, mid + 1, lo)
            hi = jnp.where(ok, hi, mid)
            return lo, hi, best

        _, _, bestb = lax.fori_loop(
            0, _BS_ITERS, bs_body,
            (i32(_THR_BITS), i32(_INF_BITS), i32(_THR_BITS)))
        thr0 = lax.bitcast_convert_type(jnp.full((_L,), bestb, i32),
                                        jnp.float32)

        def fill_body(k, carry):
            for u in range(2):
                b = (2 * k + u) * _L
                cx1[pl.ds(b, _L)] = zf
                cy1[pl.ds(b, _L)] = zf
                cx2[pl.ds(b, _L)] = zf
                cy2[pl.ds(b, _L)] = zf
                car[pl.ds(b, _L)] = zf
                csc[pl.ds(b, _L)] = ninf
                corig[pl.ds(b, _L)] = jnp.full((_L,), i32(_DEADIDX), i32)
            return carry

        lax.fori_loop(0, _CCH2, fill_body, i32(0))

        def comp_body(k, c):
            off, cbv, cbi = c
            b = k * _L
            s0 = sc[pl.ds(b, _L)]
            mask = s0 >= thr0
            cnt = plsc.all_reduce_population_count(mask)
            pos = off + plsc.cumsum(jnp.where(mask, i32(1), i32(0))) - 1
            maskw = mask & (pos < i32(_CMAX))
            idxv = b + lane
            plsc.store_scatter(csc, [pos], s0, mask=maskw)
            plsc.store_scatter(cx1, [pos], bx1[pl.ds(b, _L)], mask=maskw)
            plsc.store_scatter(cy1, [pos], by1[pl.ds(b, _L)], mask=maskw)
            plsc.store_scatter(cx2, [pos], bx2[pl.ds(b, _L)], mask=maskw)
            plsc.store_scatter(cy2, [pos], by2[pl.ds(b, _L)], mask=maskw)
            plsc.store_scatter(car, [pos], area[pl.ds(b, _L)], mask=maskw)
            plsc.store_scatter(corig, [pos], idxv, mask=maskw)
            sm = jnp.where(maskw, s0, ninf)
            upd = sm > cbv
            return (off + cnt, jnp.where(upd, sm, cbv),
                    jnp.where(upd, pos, cbi))

        offv, cbv0, cbi0 = lax.fori_loop(0, _NCH, comp_body,
                                         (zidx, ninf, zidx))
        overflow = jnp.any(offv > i32(_CMAX))

        def emit_row(t, iorig, m, valid):
            g1 = plsc.load_gather(x1, [iorig])
            g2 = plsc.load_gather(y1, [iorig])
            g3 = plsc.load_gather(x2, [iorig])
            g4 = plsc.load_gather(y2, [iorig])
            gl = plsc.load_gather(labf, [iorig])
            row = jnp.where(lane == 0, g1,
                  jnp.where(lane == 1, g2,
                  jnp.where(lane == 2, g3,
                  jnp.where(lane == 3, g4,
                  jnp.where(lane == 4, m,
                  jnp.where(lane == 5, gl, jnp.float32(0.0)))))))
            row = jnp.where(valid, row, jnp.float32(0.0))
            plsc.store_scatter(outbuf, [t * 8 + lane], row, mask=lane < 8)

        def famax_body(k, c):
            fbv, fbi = c
            for u in range(2):
                b = (2 * k + u) * _L
                s0 = sc[pl.ds(b, _L)]
                upd = s0 > fbv
                fbv = jnp.where(upd, s0, fbv)
                fbi = jnp.where(upd, b + lane, fbi)
            return fbv, fbi

        def run_full(t0, bv, bi, nw):
            def fbody(c):
                t, bv, bi, nw = c
                m = jnp.max(bv)
                valid = m > _NEG_INF
                ii = jnp.min(jnp.where(bv == m, bi, i32(_BIG)))
                iiv = jnp.full((_L,), ii, i32)
                emit_row(t, iiv, m, valid)
                plsc.store_scatter(wlist, [jnp.full((_L,), nw, i32)], iiv,
                                   mask=valid & (lane == 0))
                nw = nw + jnp.where(valid, i32(1), i32(0))
                sx1 = plsc.load_gather(bx1, [iiv])
                sy1 = plsc.load_gather(by1, [iiv])
                sx2 = plsc.load_gather(bx2, [iiv])
                sy2 = plsc.load_gather(by2, [iiv])
                sar = plsc.load_gather(area, [iiv])

                def body(k, c2):
                    nbv, nbi = c2
                    for u in range(2):
                        b = (2 * k + u) * _L
                        s0 = sc[pl.ds(b, _L)]
                        xx1 = jnp.maximum(sx1, bx1[pl.ds(b, _L)])
                        yy1 = jnp.maximum(sy1, by1[pl.ds(b, _L)])
                        xx2 = jnp.minimum(sx2, bx2[pl.ds(b, _L)])
                        yy2 = jnp.minimum(sy2, by2[pl.ds(b, _L)])
                        w = jnp.maximum(xx2 - xx1, jnp.float32(0.0))
                        h = jnp.maximum(yy2 - yy1, jnp.float32(0.0))
                        inter = w * h
                        iou = inter / (sar + area[pl.ds(b, _L)] - inter
                                       + jnp.float32(1e-9))
                        s0n = jnp.where(iou > _IOU_THR, _NEG_INF, s0)
                        sc[pl.ds(b, _L)] = s0n
                        upd = s0n > nbv
                        nbv = jnp.where(upd, s0n, nbv)
                        nbi = jnp.where(upd, b + lane, nbi)
                    return nbv, nbi

                nbv, nbi = lax.fori_loop(0, _NCH2, body, (ninf, zidx))
                return t + 1, nbv, nbi, nw

            return lax.while_loop(lambda c: c[0] < i32(_MAXOUT), fbody,
                                  (t0, bv, bi, nw))

        def run_compact(_):
            def cbody(c):
                t, bv, bi, nw = c
                m = jnp.max(bv)
                ii = jnp.min(jnp.where(bv == m, bi, i32(_BIG)))
                iiv = jnp.full((_L,), ii, i32)
                iorig = plsc.load_gather(corig, [iiv])
                emit_row(t, iorig, m, True)
                plsc.store_scatter(wlist, [jnp.full((_L,), nw, i32)],
                                   iorig, mask=lane == 0)
                nw = nw + i32(1)
                sx1 = plsc.load_gather(cx1, [iiv])
                sy1 = plsc.load_gather(cy1, [iiv])
                sx2 = plsc.load_gather(cx2, [iiv])
                sy2 = plsc.load_gather(cy2, [iiv])
                sar = plsc.load_gather(car, [iiv])

                def body(k, c2):
                    nbv, nbi = c2
                    for u in range(2):
                        b = (2 * k + u) * _L
                        s0 = csc[pl.ds(b, _L)]
                        xx1 = jnp.maximum(sx1, cx1[pl.ds(b, _L)])
                        yy1 = jnp.maximum(sy1, cy1[pl.ds(b, _L)])
                        xx2 = jnp.minimum(sx2, cx2[pl.ds(b, _L)])
                        yy2 = jnp.minimum(sy2, cy2[pl.ds(b, _L)])
                        w = jnp.maximum(xx2 - xx1, jnp.float32(0.0))
                        h = jnp.maximum(yy2 - yy1, jnp.float32(0.0))
                        inter = w * h
                        iou = inter / (sar + car[pl.ds(b, _L)] - inter
                                       + jnp.float32(1e-9))
                        s0n = jnp.where(iou > _IOU_THR, _NEG_INF, s0)
                        csc[pl.ds(b, _L)] = s0n
                        upd = s0n > nbv
                        nbv = jnp.where(upd, s0n, nbv)
                        nbi = jnp.where(upd, b + lane, nbi)
                    return nbv, nbi

                nbv, nbi = lax.fori_loop(0, _CCH2, body, (ninf, zidx))
                return t + 1, nbv, nbi, nw

            t, bv, bi, nw = lax.while_loop(
                lambda c: (c[0] < i32(_MAXOUT)) & jnp.any(c[1] > _NEG_INF),
                cbody, (i32(0), cbv0, cbi0, i32(0)))

            def finish(_):
                def replay_w(j, carry2):
                    wi = plsc.load_gather(wlist, [jnp.full((_L,), j, i32)])
                    wx1 = plsc.load_gather(bx1, [wi])
                    wy1 = plsc.load_gather(by1, [wi])
                    wx2 = plsc.load_gather(bx2, [wi])
                    wy2 = plsc.load_gather(by2, [wi])
                    war = plsc.load_gather(area, [wi])

                    def kbody(k, carry3):
                        for u in range(2):
                            b = (2 * k + u) * _L
                            s0 = sc[pl.ds(b, _L)]
                            xx1 = jnp.maximum(wx1, bx1[pl.ds(b, _L)])
                            yy1 = jnp.maximum(wy1, by1[pl.ds(b, _L)])
                            xx2 = jnp.minimum(wx2, bx2[pl.ds(b, _L)])
                            yy2 = jnp.minimum(wy2, by2[pl.ds(b, _L)])
                            w = jnp.maximum(xx2 - xx1, jnp.float32(0.0))
                            h = jnp.maximum(yy2 - yy1, jnp.float32(0.0))
                            inter = w * h
                            iou = inter / (war + area[pl.ds(b, _L)] - inter
                                           + jnp.float32(1e-9))
                            sc[pl.ds(b, _L)] = jnp.where(iou > _IOU_THR,
                                                         _NEG_INF, s0)
                        return carry3

                    lax.fori_loop(0, _NCH2, kbody, i32(0))
                    return carry2

                lax.fori_loop(0, nw, replay_w, i32(0))
                fbv, fbi = lax.fori_loop(0, _NCH2, famax_body, (ninf, zidx))
                return run_full(t, fbv, fbi, nw)

            return lax.cond(t < i32(_MAXOUT), finish, lambda _: (t, bv, bi, nw), 0)

        def run_full_from_start(_):
            fbv, fbi = lax.fori_loop(0, _NCH2, famax_body, (ninf, zidx))
            return run_full(i32(0), fbv, fbi, i32(0))

        lax.cond(overflow, run_full_from_start, run_compact, 0)
        pltpu.sync_copy(outbuf, outh.at[img])


def kernel(boxes, scores, labels):
    pad = _NP - _N

    def padr(a, v):
        return jnp.pad(a, ((0, 0), (0, pad)), constant_values=v)

    x1p = padr(boxes[..., 0], 0.0)
    y1p = padr(boxes[..., 1], 0.0)
    x2p = padr(boxes[..., 2], 0.0)
    y2p = padr(boxes[..., 3], 0.0)
    scp = padr(scores, 0.0)
    labp = padr(labels.astype(jnp.float32), 0.0)
    out = _nms_sc(x1p, y1p, x2p, y2p, scp, labp)
    return out.reshape(_B, _MAXOUT, 8)[:, :, :6]
